# baseline (speedup 1.0000x reference)
import jax
import jax.numpy as jnp
from jax.experimental import pallas as pl
from jax.experimental.pallas import tpu as pltpu

EMBD = 128
B = 16384
R8 = B // 8
BLOCK_G = 512
NDMA = 16
CHUNK = B // NDMA


def _td_copy(t_hbm, t_vmem, sems, k):
    return pltpu.make_async_copy(
        t_hbm.at[pl.ds(k * CHUNK, CHUNK), :],
        t_vmem.at[pl.ds(k * CHUNK, CHUNK), :],
        sems.at[k],
    )


def _depie_body(u_ref, t_hbm, w_ref, b_ref, o_ref, t_vmem, sems):
    pid = pl.program_id(0)

    @pl.when(pid == 0)
    def _():
        for k in range(NDMA):
            _td_copy(t_hbm, t_vmem, sems, k).start()
        for k in range(NDMA):
            _td_copy(t_hbm, t_vmem, sems, k).wait()

    t = t_vmem[pl.ds(pid * BLOCK_G * 8, BLOCK_G * 8), :].reshape(BLOCK_G, 8, 1)
    coef = t * w_ref[...] + (b_ref[...] + 1.0)
    o_ref[...] = u_ref[...] * coef


@jax.jit
def _depie_tc(user3, timediffs, w3, b3):
    grid = (R8 // BLOCK_G,)
    return pl.pallas_call(
        _depie_body,
        grid=grid,
        in_specs=[
            pl.BlockSpec((BLOCK_G, 8, EMBD), lambda i: (i, 0, 0)),
            pl.BlockSpec(memory_space=pltpu.MemorySpace.HBM),
            pl.BlockSpec((1, 1, EMBD), lambda i: (0, 0, 0)),
            pl.BlockSpec((1, 1, EMBD), lambda i: (0, 0, 0)),
        ],
        out_specs=pl.BlockSpec((BLOCK_G, 8, EMBD), lambda i: (i, 0, 0)),
        out_shape=jax.ShapeDtypeStruct((R8, 8, EMBD), jnp.float32),
        scratch_shapes=[
            pltpu.VMEM((B, 1), jnp.float32),
            pltpu.SemaphoreType.DMA((NDMA,)),
        ],
        compiler_params=pltpu.CompilerParams(
            dimension_semantics=("arbitrary",),
        ),
    )(user3, timediffs, w3, b3)


def kernel(user_embd, item_embd, timediffs, W_embd, b_embd):
    del item_embd
    user3 = user_embd.reshape(R8, 8, EMBD)
    w3 = W_embd.reshape(1, 1, EMBD)
    b3 = b_embd.reshape(1, 1, EMBD)
    out3 = _depie_tc(user3, timediffs, w3, b3)
    return out3.reshape(B, EMBD)


# speedup vs baseline: 1.4188x; 1.4188x over previous
import jax
import jax.numpy as jnp
from jax.experimental import pallas as pl
from jax.experimental.pallas import tpu as pltpu

EMBD = 128
B = 16384
R8 = B // 8
BLOCK_G = 512


def _depie_body(u_ref, t_ref, w_ref, b_ref, o_ref):
    t = t_ref[...][:, :, None]
    coef = t * w_ref[...] + (b_ref[...] + 1.0)
    o_ref[...] = u_ref[...] * coef


@jax.jit
def _depie_tc(user3, td2, w3, b3):
    grid = (R8 // BLOCK_G,)
    return pl.pallas_call(
        _depie_body,
        grid=grid,
        in_specs=[
            pl.BlockSpec((BLOCK_G, 8, EMBD), lambda i: (i, 0, 0)),
            pl.BlockSpec((BLOCK_G, 8), lambda i: (i, 0)),
            pl.BlockSpec((1, 1, EMBD), lambda i: (0, 0, 0)),
            pl.BlockSpec((1, 1, EMBD), lambda i: (0, 0, 0)),
        ],
        out_specs=pl.BlockSpec((BLOCK_G, 8, EMBD), lambda i: (i, 0, 0)),
        out_shape=jax.ShapeDtypeStruct((R8, 8, EMBD), jnp.float32),
        compiler_params=pltpu.CompilerParams(
            dimension_semantics=("arbitrary",),
        ),
    )(user3, td2, w3, b3)


def kernel(user_embd, item_embd, timediffs, W_embd, b_embd):
    del item_embd
    user3 = user_embd.reshape(R8, 8, EMBD)
    td2 = jnp.sum(timediffs.reshape(R8, 8, 1), axis=2)
    w3 = W_embd.reshape(1, 1, EMBD)
    b3 = b_embd.reshape(1, 1, EMBD)
    out3 = _depie_tc(user3, td2, w3, b3)
    return out3.reshape(B, EMBD)


# speedup vs baseline: 1.5824x; 1.1153x over previous
import jax
import jax.numpy as jnp
from jax.experimental import pallas as pl
from jax.experimental.pallas import tpu as pltpu

EMBD = 128
B = 16384
R8 = B // 8
BLOCK_G = 1024


def _depie_body(u_ref, t_ref, w_ref, b_ref, o_ref):
    t = t_ref[...][:, :, None]
    coef = t * w_ref[...] + (b_ref[...] + 1.0)
    o_ref[...] = u_ref[...] * coef


@jax.jit
def _depie_tc(user3, td2, w3, b3):
    grid = (R8 // BLOCK_G,)
    return pl.pallas_call(
        _depie_body,
        grid=grid,
        in_specs=[
            pl.BlockSpec((BLOCK_G, 8, EMBD), lambda i: (i, 0, 0)),
            pl.BlockSpec((BLOCK_G, 8), lambda i: (i, 0)),
            pl.BlockSpec((1, 1, EMBD), lambda i: (0, 0, 0)),
            pl.BlockSpec((1, 1, EMBD), lambda i: (0, 0, 0)),
        ],
        out_specs=pl.BlockSpec((BLOCK_G, 8, EMBD), lambda i: (i, 0, 0)),
        out_shape=jax.ShapeDtypeStruct((R8, 8, EMBD), jnp.float32),
        compiler_params=pltpu.CompilerParams(
            dimension_semantics=("arbitrary",),
        ),
    )(user3, td2, w3, b3)


def kernel(user_embd, item_embd, timediffs, W_embd, b_embd):
    del item_embd
    user3 = user_embd.reshape(R8, 8, EMBD)
    td2 = jnp.sum(timediffs.reshape(R8, 8, 1), axis=2)
    w3 = W_embd.reshape(1, 1, EMBD)
    b3 = b_embd.reshape(1, 1, EMBD)
    out3 = _depie_tc(user3, td2, w3, b3)
    return out3.reshape(B, EMBD)


# speedup vs baseline: 1.5873x; 1.0031x over previous
import jax
import jax.numpy as jnp
from jax.experimental import pallas as pl
from jax.experimental.pallas import tpu as pltpu

EMBD = 128
B = 16384
R8 = B // 8
BLOCK_G = 1024


def _depie_body(u_ref, t_ref, w_ref, b_ref, o_ref):
    t = t_ref[...][:, :, None]
    coef = t * w_ref[...] + (b_ref[...] + 1.0)
    o_ref[...] = u_ref[...] * coef


@jax.jit
def _depie_tc(user3, td2, w3, b3):
    grid = (R8 // BLOCK_G,)
    return pl.pallas_call(
        _depie_body,
        grid=grid,
        in_specs=[
            pl.BlockSpec((BLOCK_G, 8, EMBD), lambda i: (i, 0, 0)),
            pl.BlockSpec((BLOCK_G, 8), lambda i: (i, 0)),
            pl.BlockSpec((1, 1, EMBD), lambda i: (0, 0, 0)),
            pl.BlockSpec((1, 1, EMBD), lambda i: (0, 0, 0)),
        ],
        out_specs=pl.BlockSpec((BLOCK_G, 8, EMBD), lambda i: (i, 0, 0)),
        out_shape=jax.ShapeDtypeStruct((R8, 8, EMBD), jnp.float32),
        compiler_params=pltpu.CompilerParams(
            dimension_semantics=("arbitrary",),
        ),
    )(user3, td2, w3, b3)


def kernel(user_embd, item_embd, timediffs, W_embd, b_embd):
    del item_embd
    user3 = user_embd.reshape(R8, 8, EMBD)
    td2 = timediffs[:, 0].reshape(R8, 8)
    w3 = W_embd.reshape(1, 1, EMBD)
    b3 = b_embd.reshape(1, 1, EMBD)
    out3 = _depie_tc(user3, td2, w3, b3)
    return out3.reshape(B, EMBD)
